# baseline (speedup 1.0000x reference)
import functools

import jax
import jax.numpy as jnp
from jax import lax
from jax.experimental import pallas as pl
from jax.experimental.pallas import tpu as pltpu
from jax.experimental.pallas import tpu_sc as plsc

_NUM_WORKERS = 32
_CHUNK = 512
_NBUF = 4
_VPC = _CHUNK // 16


def _lookup(idx, W, N, D):
    b_per_w = N // _NUM_WORKERS
    n_chunks = b_per_w // _CHUNK
    mesh = plsc.VectorSubcoreMesh(core_axis_name="c", subcore_axis_name="s")

    @functools.partial(
        pl.kernel,
        mesh=mesh,
        out_type=jax.ShapeDtypeStruct((N, D), jnp.float32),
        scratch_types=[
            pltpu.VMEM_SHARED((1000, 32), jnp.float32),
            pltpu.VMEM((2, 32, 1, 512), jnp.int32),
            pltpu.VMEM((_CHUNK,), jnp.int32),
            pltpu.VMEM((_CHUNK,), jnp.int32),
            pltpu.VMEM((_NBUF, _CHUNK), jnp.int32),
            pltpu.VMEM((_NBUF, _CHUNK, D), jnp.float32),
            [pltpu.SemaphoreType.DMA] * _NBUF,
            [pltpu.SemaphoreType.DMA] * _NBUF,
        ],
        compiler_params=pltpu.CompilerParams(
            use_tc_tiling_on_sc=False, needs_layout_passes=False),
    )
    def k(table_hbm, idx_hbm, out_hbm, table_v, idx_v, pat_bt, pat_p, idx_p,
          rows_v, gsems, ssems):
        wid = lax.axis_index("s") * 2 + lax.axis_index("c")
        base = wid * b_per_w

        @pl.when(lax.axis_index("s") == 0)
        def _():
            pltpu.sync_copy(table_hbm, table_v)
        pltpu.sync_copy(idx_hbm.at[pl.ds(wid * 2, 2)], idx_v)

        def mkpat(v, carry):
            j = lax.iota(jnp.int32, 16) + v * 16
            pat_bt[pl.ds(v * 16, 16)] = (((j >> 8) & 3) << 3) | ((j >> 2) & 7)
            pat_p[pl.ds(v * 16, 16)] = (((j >> 5) & 7) << 2) | (j & 3)
            return carry
        lax.fori_loop(0, _VPC, mkpat, 0)

        def permute(g, b):
            def body(v, carry):
                bt = pat_bt[pl.ds(v * 16, 16)] + g * (_CHUNK // 32)
                p = pat_p[pl.ds(v * 16, 16)]
                idx_p[b, pl.ds(v * 16, 16)] = plsc.load_gather(
                    idx_v,
                    [bt >> 9, p, jnp.zeros((16,), jnp.int32), bt & 511])
                return carry
            lax.fori_loop(0, _VPC, body, 0)

        plsc.subcore_barrier()
        gathers = [None] * _NBUF
        stores = [None] * _NBUF
        for q in range(2):
            permute(q, q)
            gathers[q] = pltpu.async_copy(
                table_v.at[idx_p.at[q]], rows_v.at[q], gsems[q])
        for g in range(n_chunks):
            b = g % _NBUF
            if g + 2 < n_chunks:
                b2 = (g + 2) % _NBUF
                if stores[b2] is not None:
                    stores[b2].wait()
                permute(g + 2, b2)
                gathers[b2] = pltpu.async_copy(
                    table_v.at[idx_p.at[b2]], rows_v.at[b2], gsems[b2])
            gathers[b].wait()
            stores[b] = pltpu.async_copy(
                rows_v.at[b], out_hbm.at[pl.ds(base + g * _CHUNK, _CHUNK)],
                ssems[b])
        for q in range(_NBUF):
            if stores[q] is not None:
                stores[q].wait()

    return k(W, idx)


def kernel(spikes, W):
    bs, T, Pn, Pt = spikes.shape
    V, D = W.shape
    N = bs * T * Pn * Pt
    idx = spikes.transpose(0, 2, 3, 1)
    out = _lookup(idx, W, N, D)
    out = out.reshape(N // 8 // 32, 8, 8, 128)
    out = out.transpose(0, 2, 1, 3)
    return out.reshape(bs, T, Pn * Pt * D)


# speedup vs baseline: 1.0095x; 1.0095x over previous
import functools

import jax
import jax.numpy as jnp
from jax import lax
from jax.experimental import pallas as pl
from jax.experimental.pallas import tpu as pltpu
from jax.experimental.pallas import tpu_sc as plsc

_NUM_WORKERS = 32
_CHUNK = 1024
_VPC = _CHUNK // 16


def _lookup(idx, W, N, D):
    b_per_w = N // _NUM_WORKERS
    n_chunks = b_per_w // _CHUNK
    mesh = plsc.VectorSubcoreMesh(core_axis_name="c", subcore_axis_name="s")

    @functools.partial(
        pl.kernel,
        mesh=mesh,
        out_type=jax.ShapeDtypeStruct((N, D), jnp.float32),
        scratch_types=[
            pltpu.VMEM_SHARED((2, 1000, 32), jnp.float32),
            pltpu.VMEM((2, 32, 1, 512), jnp.int32),
            pltpu.VMEM((_CHUNK,), jnp.int32),
            pltpu.VMEM((_CHUNK,), jnp.int32),
            pltpu.VMEM((2, _CHUNK), jnp.int32),
            pltpu.VMEM((2, _CHUNK, D), jnp.float32),
            pltpu.SemaphoreType.DMA,
            pltpu.SemaphoreType.DMA,
            pltpu.SemaphoreType.DMA,
            pltpu.SemaphoreType.DMA,
        ],
        compiler_params=pltpu.CompilerParams(
            use_tc_tiling_on_sc=False, needs_layout_passes=False),
    )
    def k(table_hbm, idx_hbm, out_hbm, table_v, idx_v, pat_bt, pat_p, idx_p,
          rows_v, gs0, gs1, ss0, ss1):
        wid = lax.axis_index("s") * 2 + lax.axis_index("c")
        base = wid * b_per_w

        sid = lax.axis_index("s")

        @pl.when(sid < 2)
        def _():
            pltpu.sync_copy(table_hbm, table_v.at[sid])
        pltpu.sync_copy(idx_hbm.at[pl.ds(wid * 2, 2)], idx_v)

        def mkpat(v, carry):
            j = lax.iota(jnp.int32, 16) + v * 16
            pat_bt[pl.ds(v * 16, 16)] = (((j >> 8) & 3) << 3) | ((j >> 2) & 7)
            pat_p[pl.ds(v * 16, 16)] = (((j >> 5) & 7) << 2) | (j & 3)
            return carry
        lax.fori_loop(0, _VPC, mkpat, 0)

        def permute(g, b):
            def body(v, carry):
                bt = pat_bt[pl.ds(v * 16, 16)] + g * 32
                p = pat_p[pl.ds(v * 16, 16)]
                idx_p[b, pl.ds(v * 16, 16)] = plsc.load_gather(
                    idx_v,
                    [bt >> 9, p, jnp.zeros((16,), jnp.int32), bt & 511])
                return carry
            lax.fori_loop(0, _VPC, body, 0)

        plsc.subcore_barrier()
        gsems = (gs0, gs1)
        ssems = (ss0, ss1)
        stores = [None, None]
        permute(0, 0)
        for g in range(n_chunks):
            b = g % 2
            if stores[b] is not None:
                stores[b].wait()
            gather = pltpu.async_copy(
                table_v.at[sid & 1].at[idx_p.at[b]], rows_v.at[b], gsems[b])
            if g + 1 < n_chunks:
                permute(g + 1, 1 - b)
            gather.wait()
            stores[b] = pltpu.async_copy(
                rows_v.at[b], out_hbm.at[pl.ds(base + g * _CHUNK, _CHUNK)],
                ssems[b])
        for b in (0, 1):
            stores[b].wait()

    return k(W, idx)


def kernel(spikes, W):
    bs, T, Pn, Pt = spikes.shape
    V, D = W.shape
    N = bs * T * Pn * Pt
    idx = spikes.transpose(0, 2, 3, 1)
    out = _lookup(idx, W, N, D)
    out = out.reshape(N // 8 // 32, 8, 8, 128)
    out = out.transpose(0, 2, 1, 3)
    return out.reshape(bs, T, Pn * Pt * D)


# speedup vs baseline: 1.0106x; 1.0011x over previous
import functools

import jax
import jax.numpy as jnp
from jax import lax
from jax.experimental import pallas as pl
from jax.experimental.pallas import tpu as pltpu
from jax.experimental.pallas import tpu_sc as plsc

_NUM_WORKERS = 32
_CHUNK = 1024
_VPC = _CHUNK // 16


def _lookup(idx, W, N, D):
    b_per_w = N // _NUM_WORKERS
    n_chunks = b_per_w // _CHUNK
    mesh = plsc.VectorSubcoreMesh(core_axis_name="c", subcore_axis_name="s")

    @functools.partial(
        pl.kernel,
        mesh=mesh,
        out_type=jax.ShapeDtypeStruct((N, D), jnp.float32),
        scratch_types=[
            pltpu.VMEM_SHARED((1000, 32), jnp.float32),
            pltpu.VMEM((2, 32, 1, 512), jnp.int32),
            pltpu.VMEM((_CHUNK,), jnp.int32),
            pltpu.VMEM((_CHUNK,), jnp.int32),
            pltpu.VMEM((2, _CHUNK), jnp.int32),
            pltpu.VMEM((2, _CHUNK, D), jnp.float32),
            pltpu.SemaphoreType.DMA,
            pltpu.SemaphoreType.DMA,
            pltpu.SemaphoreType.DMA,
            pltpu.SemaphoreType.DMA,
        ],
        compiler_params=pltpu.CompilerParams(
            use_tc_tiling_on_sc=False, needs_layout_passes=False),
    )
    def k(table_hbm, idx_hbm, out_hbm, table_v, idx_v, pat_bt, pat_p, idx_p,
          rows_v, gs0, gs1, ss0, ss1):
        wid = lax.axis_index("s") * 2 + lax.axis_index("c")
        base = wid * b_per_w

        @pl.when(lax.axis_index("s") == 0)
        def _():
            pltpu.sync_copy(table_hbm, table_v)
        pltpu.sync_copy(idx_hbm.at[pl.ds(wid * 2, 2)], idx_v)

        def mkpat(v, carry):
            j = lax.iota(jnp.int32, 16) + v * 16
            pat_bt[pl.ds(v * 16, 16)] = (((j >> 8) & 3) << 3) | ((j >> 2) & 7)
            pat_p[pl.ds(v * 16, 16)] = (((j >> 5) & 7) << 2) | (j & 3)
            return carry
        lax.fori_loop(0, _VPC, mkpat, 0)

        def permute(g, b):
            def body(v, carry):
                bt = pat_bt[pl.ds(v * 16, 16)] + g * 32
                p = pat_p[pl.ds(v * 16, 16)]
                idx_p[b, pl.ds(v * 16, 16)] = plsc.load_gather(
                    idx_v,
                    [bt >> 9, p, jnp.zeros((16,), jnp.int32), bt & 511])
                return carry
            lax.fori_loop(0, _VPC, body, 0)

        plsc.subcore_barrier()
        gsems = (gs0, gs1)
        ssems = (ss0, ss1)
        stores = [None, None]
        permute(0, 0)
        for g in range(n_chunks):
            b = g % 2
            if stores[b] is not None:
                stores[b].wait()
            gather = pltpu.async_copy(
                table_v.at[idx_p.at[b]], rows_v.at[b], gsems[b])
            if g + 1 < n_chunks:
                permute(g + 1, 1 - b)
            gather.wait()
            stores[b] = pltpu.async_copy(
                rows_v.at[b], out_hbm.at[pl.ds(base + g * _CHUNK, _CHUNK)],
                ssems[b])
        for b in (0, 1):
            stores[b].wait()

    return k(W, idx)


def kernel(spikes, W):
    bs, T, Pn, Pt = spikes.shape
    V, D = W.shape
    N = bs * T * Pn * Pt
    idx = spikes.transpose(0, 2, 3, 1)
    out = _lookup(idx, W, N, D)
    out = out.reshape(N // 8 // 32, 8, 8, 128)
    out = out.transpose(0, 2, 1, 3)
    return out.reshape(bs, T, Pn * Pt * D)
